# baseline (speedup 1.0000x reference)
import jax
import jax.numpy as jnp
from jax.experimental import pallas as pl


def _silu(x):
    return x * jax.nn.sigmoid(x)


def _copy_body(x_ref, o_ref):
    o_ref[...] = x_ref[...]


def kernel(q, mu, edge_index, rbf, unit_vectors, cutoff_values, W1, b1, W2, b2, Wf1, bf1, Wf2, bf2, Wv, Wm1, bm1, Wm2, bm2):
    num_nodes = q.shape[0]
    source = edge_index[1]
    target = edge_index[0]
    filters = (_silu(rbf @ Wf1 + bf1) @ Wf2 + bf2) * cutoff_values[:, None]
    filter_q, filter_r, filter_mu = jnp.split(filters, 3, axis=-1)
    x = _silu(q @ W1 + b1) @ W2 + b2
    x_q, x_r, x_mu = jnp.split(x, 3, axis=-1)
    x_q_src = x_q[source] * filter_q
    scalar_msg = jax.ops.segment_sum(x_q_src, target, num_segments=num_nodes)
    x_r_src = x_r[source] * filter_r
    x_mu_src = x_mu[source] * filter_mu
    mu_src = mu[source]
    vec_new = unit_vectors[:, :, None] * x_r_src[:, None, :]
    vec_prop = mu_src * x_mu_src[:, None, :]
    vector_msg = jax.ops.segment_sum(vec_new + vec_prop, target, num_segments=num_nodes)
    deg = jnp.clip(jnp.bincount(target, length=num_nodes), 1, None).astype(q.dtype)
    scalar_msg = scalar_msg / deg[:, None]
    vector_msg = vector_msg / deg[:, None, None]
    q = q + scalar_msg
    mu = mu + vector_msg
    mu_cat = jnp.einsum('ncf,fo->nco', mu, Wv)
    mu_v, mu_w = jnp.split(mu_cat, 2, axis=-1)
    mu_v_norm = jnp.sqrt(jnp.sum(mu_v ** 2, axis=1) + 1e-08)
    scalar_input = jnp.concatenate([q, mu_v_norm], axis=-1)
    delta = _silu(scalar_input @ Wm1 + bm1) @ Wm2 + bm2
    dq, dmu_scale, dqmu = jnp.split(delta, 3, axis=-1)
    inner = jnp.sum(mu_v * mu_w, axis=1)
    q = q + dq + dqmu * inner
    q = pl.pallas_call(
        _copy_body,
        out_shape=jax.ShapeDtypeStruct(q.shape, q.dtype),
    )(q)
    mu = mu + mu_w * dmu_scale[:, None, :]
    return q, mu


# speedup vs baseline: 6.3339x; 6.3339x over previous
import functools

import jax
import jax.numpy as jnp
from jax import lax
from jax.experimental import pallas as pl
from jax.experimental.pallas import tpu as pltpu
from jax.experimental.pallas import tpu_sc as plsc


H = 128
HH = 64
B = 128
K = 8
NTILES = 16


def _silu(x):
    return x * jax.nn.sigmoid(x)


def _node_body(q_ref, W1_ref, b1_ref, W2_ref, b2_ref,
               xql_ref, xqh_ref, xrml_ref, xrmh_ref):
    h = _silu(jnp.dot(q_ref[...], W1_ref[...], preferred_element_type=jnp.float32)
              + b1_ref[...])
    x = jnp.dot(h, W2_ref[...], preferred_element_type=jnp.float32) + b2_ref[...]
    xql_ref[...] = x[:, 0:HH]
    xqh_ref[...] = x[:, HH:H]
    xrml_ref[...] = jnp.concatenate([x[:, H:H + HH], x[:, 2 * H:2 * H + HH]], axis=1)
    xrmh_ref[...] = jnp.concatenate([x[:, H + HH:2 * H], x[:, 2 * H + HH:3 * H]], axis=1)


def _node_mlp(q, W1, b1, W2, b2, bn):
    n = q.shape[0]
    full = lambda a: pl.BlockSpec(a.shape, lambda i: (0,) * a.ndim)
    half = jax.ShapeDtypeStruct((n, HH), jnp.float32)
    pair = jax.ShapeDtypeStruct((n, H), jnp.float32)
    return pl.pallas_call(
        _node_body,
        grid=(n // bn,),
        in_specs=[
            pl.BlockSpec((bn, H), lambda i: (i, 0)),
            full(W1), full(b1), full(W2), full(b2),
        ],
        out_specs=[
            pl.BlockSpec((bn, HH), lambda i: (i, 0)),
            pl.BlockSpec((bn, HH), lambda i: (i, 0)),
            pl.BlockSpec((bn, H), lambda i: (i, 0)),
            pl.BlockSpec((bn, H), lambda i: (i, 0)),
        ],
        out_shape=[half, half, pair, pair],
    )(q, W1, b1, W2, b2)


def _filter_body(rbf_ref, cut_ref, uv_ref, Wf1_ref, bf1_ref, Wf2_ref, bf2_ref,
                 fql_ref, fqh_ref, g0l_ref, g0h_ref, g1l_ref, g1h_ref,
                 g2l_ref, g2h_ref):
    h = _silu(jnp.dot(rbf_ref[...], Wf1_ref[...], preferred_element_type=jnp.float32)
              + bf1_ref[...])
    f = (jnp.dot(h, Wf2_ref[...], preferred_element_type=jnp.float32)
         + bf2_ref[...]) * cut_ref[...]
    fql_ref[...] = f[:, 0:HH]
    fqh_ref[...] = f[:, HH:H]
    frl = f[:, H:H + HH]
    frh = f[:, H + HH:2 * H]
    fml = f[:, 2 * H:2 * H + HH]
    fmh = f[:, 2 * H + HH:3 * H]
    for c, (gl_ref, gh_ref) in enumerate(((g0l_ref, g0h_ref),
                                          (g1l_ref, g1h_ref),
                                          (g2l_ref, g2h_ref))):
        uvc = uv_ref[:, c:c + 1]
        gl_ref[...] = jnp.concatenate([frl * uvc, fml], axis=1)
        gh_ref[...] = jnp.concatenate([frh * uvc, fmh], axis=1)


def _filter_mlp(rbf_p, cut_p, uv_p, Wf1, bf1, Wf2, bf2, be):
    ep, nrbf = rbf_p.shape
    full = lambda a: pl.BlockSpec(a.shape, lambda i: (0,) * a.ndim)
    half = jax.ShapeDtypeStruct((ep, HH), jnp.float32)
    pair = jax.ShapeDtypeStruct((ep, H), jnp.float32)
    return pl.pallas_call(
        _filter_body,
        grid=(ep // be,),
        in_specs=[
            pl.BlockSpec((be, nrbf), lambda i: (i, 0)),
            pl.BlockSpec((be, 1), lambda i: (i, 0)),
            pl.BlockSpec((be, 3), lambda i: (i, 0)),
            full(Wf1), full(bf1), full(Wf2), full(bf2),
        ],
        out_specs=[pl.BlockSpec((be, HH), lambda i: (i, 0))] * 2
        + [pl.BlockSpec((be, H), lambda i: (i, 0))] * 6,
        out_shape=[half, half, pair, pair, pair, pair, pair, pair],
    )(rbf_p, cut_p, uv_p, Wf1, bf1, Wf2, bf2)


def _sc_messages(src2, tgt2, xq_l, xq_h, xrm_l, xrm_h, mus_l, mus_h,
                 fq_l, fq_h, gfs_l, gfs_h, zacc, zdeg, np_rows):
    ep = src2.shape[0] * src2.shape[1]
    e_per_tile = ep // NTILES
    chunks = e_per_tile // B
    outers = chunks // K
    rpt = np_rows // NTILES

    mesh = plsc.VectorSubcoreMesh(core_axis_name="c", subcore_axis_name="s")
    acc_ty = jax.ShapeDtypeStruct((np_rows, HH), jnp.float32)
    deg_ty = jax.ShapeDtypeStruct((np_rows, 16), jnp.float32)

    @functools.partial(
        pl.kernel,
        out_type=[acc_ty, acc_ty, deg_ty,
                  acc_ty, acc_ty, acc_ty, acc_ty, acc_ty, acc_ty],
        mesh=mesh,
        scratch_types=[
            pltpu.VMEM_SHARED((np_rows, HH), jnp.float32),
            pltpu.VMEM_SHARED((np_rows, 16), jnp.float32),
            pltpu.VMEM((K, B), jnp.int32),
            pltpu.VMEM((K, B), jnp.int32),
            pltpu.VMEM((B, H), jnp.float32),
            pltpu.VMEM((B, H), jnp.float32),
            pltpu.VMEM((B, HH), jnp.float32),
            pltpu.VMEM((B, HH), jnp.float32),
            pltpu.VMEM((B, 16), jnp.float32),
            pltpu.SemaphoreType.DMA,
            pltpu.SemaphoreType.DMA,
            pltpu.SemaphoreType.DMA,
        ],
        compiler_params=pltpu.CompilerParams(use_tc_tiling_on_sc=False),
    )
    def sc_kernel(src2_h, tgt2_h, xql_h, xqh_h, xrml_h, xrmh_h,
                  mu0l_h, mu1l_h, mu2l_h, mu0h_h, mu1h_h, mu2h_h,
                  fql_h, fqh_h, gf0l_h, gf1l_h, gf2l_h, gf0h_h, gf1h_h, gf2h_h,
                  zacc_h, zdeg_h,
                  oql_h, oqh_h, odeg_h, o0l_h, o0h_h, o1l_h, o1h_h, o2l_h, o2h_h,
                  acc_sh, deg_sh, idx_s2, idx_t2, big_a, big_b, t64_a, t64_b,
                  ones, sem0, sem1, sem2):
        cid = lax.axis_index("c")
        sid = lax.axis_index("s")
        row0 = sid * rpt

        def _ones_row(r, carry):
            ones[r, :] = jnp.ones((16,), jnp.float32)
            return carry
        lax.fori_loop(0, B, _ones_row, 0)

        def load_idx(o):
            r0 = (sid * e_per_tile) // B + o * K
            pltpu.sync_copy(src2_h.at[pl.ds(r0, K)], idx_s2)
            pltpu.sync_copy(tgt2_h.at[pl.ds(r0, K)], idx_t2)

        def sweep_vec(xrm_h, mu_h, gf_h):
            def outer(o, carry):
                base = sid * e_per_tile + o * (K * B)
                load_idx(o)
                for k in range(K):
                    e0 = base + k * B
                    c1 = pltpu.async_copy(xrm_h.at[idx_s2.at[k]], big_a, sem0)
                    c2 = pltpu.async_copy(mu_h.at[idx_s2.at[k]], t64_a, sem1)
                    c3 = pltpu.async_copy(gf_h.at[pl.ds(e0, B)], big_b, sem2)
                    c1.wait(); c2.wait(); c3.wait()

                    def row(r, c):
                        for gi in range(HH // 16):
                            sl = pl.ds(gi * 16, 16)
                            sh = pl.ds(HH + gi * 16, 16)
                            t64_a[r, sl] = (big_a[r, sl] * big_b[r, sl]
                                            + t64_a[r, sl] * (big_a[r, sh] * big_b[r, sh]))
                        return c
                    lax.fori_loop(0, B, row, 0)
                    pltpu.sync_copy(t64_a, acc_sh.at[idx_t2.at[k]], add=True)
                return carry
            lax.fori_loop(0, outers, outer, 0)

        def sweep_q(xq_h, fq_h, do_deg):
            def outer(o, carry):
                base = sid * e_per_tile + o * (K * B)
                load_idx(o)
                for k in range(K):
                    e0 = base + k * B
                    c1 = pltpu.async_copy(xq_h.at[idx_s2.at[k]], t64_a, sem0)
                    c2 = pltpu.async_copy(fq_h.at[pl.ds(e0, B)], t64_b, sem1)
                    c1.wait(); c2.wait()

                    def row(r, c):
                        for gi in range(HH // 16):
                            sl = pl.ds(gi * 16, 16)
                            t64_a[r, sl] = t64_a[r, sl] * t64_b[r, sl]
                        return c
                    lax.fori_loop(0, B, row, 0)
                    pltpu.sync_copy(t64_a, acc_sh.at[idx_t2.at[k]], add=True)
                    if do_deg:
                        pltpu.sync_copy(ones, deg_sh.at[idx_t2.at[k]], add=True)
                return carry
            lax.fori_loop(0, outers, outer, 0)

        def zero_acc():
            pltpu.sync_copy(zacc_h.at[pl.ds(row0, rpt)],
                            acc_sh.at[pl.ds(row0, rpt)])

        def drain_acc(out_h):
            pltpu.sync_copy(acc_sh.at[pl.ds(row0, rpt)],
                            out_h.at[pl.ds(row0, rpt)])

        def program(xq_h, fq_h, xrm_h, mu_hs, gf_hs, oq_h, ov_hs, do_deg):
            zero_acc()
            if do_deg:
                pltpu.sync_copy(zdeg_h.at[pl.ds(row0, rpt)],
                                deg_sh.at[pl.ds(row0, rpt)])
            plsc.subcore_barrier()
            sweep_q(xq_h, fq_h, do_deg)
            plsc.subcore_barrier()
            drain_acc(oq_h)
            if do_deg:
                pltpu.sync_copy(deg_sh.at[pl.ds(row0, rpt)],
                                odeg_h.at[pl.ds(row0, rpt)])
            for c in range(3):
                zero_acc()
                plsc.subcore_barrier()
                sweep_vec(xrm_h, mu_hs[c], gf_hs[c])
                plsc.subcore_barrier()
                drain_acc(ov_hs[c])

        @pl.when(cid == 0)
        def _():
            program(xql_h, fql_h, xrml_h, (mu0l_h, mu1l_h, mu2l_h),
                    (gf0l_h, gf1l_h, gf2l_h), oql_h, (o0l_h, o1l_h, o2l_h),
                    do_deg=True)

        @pl.when(cid == 1)
        def _():
            program(xqh_h, fqh_h, xrmh_h, (mu0h_h, mu1h_h, mu2h_h),
                    (gf0h_h, gf1h_h, gf2h_h), oqh_h, (o0h_h, o1h_h, o2h_h),
                    do_deg=False)

    return sc_kernel(src2, tgt2, xq_l, xq_h, xrm_l, xrm_h,
                     mus_l[0], mus_l[1], mus_l[2], mus_h[0], mus_h[1], mus_h[2],
                     fq_l, fq_h, gfs_l[0], gfs_l[1], gfs_l[2],
                     gfs_h[0], gfs_h[1], gfs_h[2], zacc, zdeg)


def _final_body(q_ref, mu_ref, aql_ref, aqh_ref, adeg_ref,
                a0l_ref, a0h_ref, a1l_ref, a1h_ref, a2l_ref, a2h_ref,
                Wv_ref, Wm1_ref, bm1_ref, Wm2_ref, bm2_ref,
                qo_ref, muo_ref):
    deg = jnp.clip(adeg_ref[:, 0:1], 1.0, None)
    inv = 1.0 / deg
    qn = q_ref[...] + jnp.concatenate([aql_ref[...], aqh_ref[...]], axis=1) * inv
    avs = ((a0l_ref[...], a0h_ref[...]), (a1l_ref[...], a1h_ref[...]),
           (a2l_ref[...], a2h_ref[...]))
    mun = [mu_ref[:, c, :] + jnp.concatenate(avs[c], axis=1) * inv
           for c in range(3)]
    mc = [jnp.dot(m, Wv_ref[...], preferred_element_type=jnp.float32) for m in mun]
    mu_v = [m[:, :H] for m in mc]
    mu_w = [m[:, H:] for m in mc]
    nrm = jnp.sqrt(mu_v[0] ** 2 + mu_v[1] ** 2 + mu_v[2] ** 2 + 1e-08)
    si = jnp.concatenate([qn, nrm], axis=-1)
    delta = (jnp.dot(_silu(jnp.dot(si, Wm1_ref[...], preferred_element_type=jnp.float32)
                           + bm1_ref[...]),
                     Wm2_ref[...], preferred_element_type=jnp.float32) + bm2_ref[...])
    dq = delta[:, :H]
    dmu = delta[:, H:2 * H]
    dqmu = delta[:, 2 * H:]
    inner = mu_v[0] * mu_w[0] + mu_v[1] * mu_w[1] + mu_v[2] * mu_w[2]
    qo_ref[...] = qn + dq + dqmu * inner
    muo_ref[...] = jnp.stack([mun[c] + mu_w[c] * dmu for c in range(3)], axis=1)


def _final_mix(q, mu, aq_l, aq_h, adeg, avs_l, avs_h, Wv, Wm1, bm1, Wm2, bm2, bn):
    n = q.shape[0]
    full = lambda a: pl.BlockSpec(a.shape, lambda i: (0,) * a.ndim)
    row = lambda w: pl.BlockSpec((bn, w), lambda i: (i, 0))
    return pl.pallas_call(
        _final_body,
        grid=(n // bn,),
        in_specs=[
            row(H),
            pl.BlockSpec((bn, 3, H), lambda i: (i, 0, 0)),
            row(HH), row(HH), row(16),
            row(HH), row(HH), row(HH), row(HH), row(HH), row(HH),
            full(Wv), full(Wm1), full(bm1), full(Wm2), full(bm2),
        ],
        out_specs=[
            row(H),
            pl.BlockSpec((bn, 3, H), lambda i: (i, 0, 0)),
        ],
        out_shape=[
            jax.ShapeDtypeStruct((n, H), jnp.float32),
            jax.ShapeDtypeStruct((n, 3, H), jnp.float32),
        ],
    )(q, mu, aq_l, aq_h, adeg, avs_l[0], avs_h[0], avs_l[1], avs_h[1],
      avs_l[2], avs_h[2], Wv, Wm1, bm1, Wm2, bm2)


def kernel(q, mu, edge_index, rbf, unit_vectors, cutoff_values,
           W1, b1, W2, b2, Wf1, bf1, Wf2, bf2, Wv, Wm1, bm1, Wm2, bm2):
    n = q.shape[0]
    e = edge_index.shape[1]

    ew = NTILES * B * K
    ep = ((e + ew - 1) // ew) * ew
    np_rows = ((n + 8) + NTILES * 8 - 1) // (NTILES * 8) * (NTILES * 8)

    source = edge_index[1]
    target = edge_index[0]
    pad = ep - e
    src2 = jnp.concatenate([source, jnp.zeros((pad,), jnp.int32)]).reshape(ep // B, B)
    tgt2 = jnp.concatenate([target, jnp.full((pad,), n, jnp.int32)]).reshape(ep // B, B)
    rbf_p = jnp.pad(rbf, ((0, pad), (0, 0)))
    cut_p = jnp.pad(cutoff_values, (0, pad))[:, None]
    uv_p = jnp.pad(unit_vectors, ((0, pad), (0, 0)))
    mu_t = jnp.transpose(mu, (1, 0, 2))
    mus_l = tuple(mu_t[c, :, :HH] for c in range(3))
    mus_h = tuple(mu_t[c, :, HH:] for c in range(3))
    zacc = jnp.zeros((np_rows, HH), jnp.float32)
    zdeg = jnp.zeros((np_rows, 16), jnp.float32)

    b1r, b2r = b1[None, :], b2[None, :]
    bf1r, bf2r = bf1[None, :], bf2[None, :]
    bm1r, bm2r = bm1[None, :], bm2[None, :]

    xq_l, xq_h, xrm_l, xrm_h = _node_mlp(q, W1, b1r, W2, b2r, bn=1000)
    fq_l, fq_h, gf0_l, gf0_h, gf1_l, gf1_h, gf2_l, gf2_h = _filter_mlp(
        rbf_p, cut_p, uv_p, Wf1, bf1r, Wf2, bf2r, be=1024)
    aq_l, aq_h, adeg, a0_l, a0_h, a1_l, a1_h, a2_l, a2_h = _sc_messages(
        src2, tgt2, xq_l, xq_h, xrm_l, xrm_h, mus_l, mus_h,
        fq_l, fq_h, (gf0_l, gf1_l, gf2_l), (gf0_h, gf1_h, gf2_h),
        zacc, zdeg, np_rows)
    return _final_mix(q, mu, aq_l, aq_h, adeg, (a0_l, a1_l, a2_l),
                      (a0_h, a1_h, a2_h), Wv, Wm1, bm1r, Wm2, bm2r, bn=1000)


# speedup vs baseline: 7.5550x; 1.1928x over previous
import functools

import jax
import jax.numpy as jnp
from jax import lax
from jax.experimental import pallas as pl
from jax.experimental.pallas import tpu as pltpu
from jax.experimental.pallas import tpu_sc as plsc


H = 128
HH = 64
B = 128
K = 4
NTILES = 16


def _silu(x):
    return x * jax.nn.sigmoid(x)


def _node_body(q_ref, W1_ref, b1_ref, W2_ref, b2_ref,
               xq_ref, xrml_ref, xrmh_ref):
    h = _silu(jnp.dot(q_ref[...], W1_ref[...], preferred_element_type=jnp.float32)
              + b1_ref[...])
    x = jnp.dot(h, W2_ref[...], preferred_element_type=jnp.float32) + b2_ref[...]
    xq_ref[...] = x[:, 0:H]
    xrml_ref[...] = jnp.concatenate([x[:, H:H + HH], x[:, 2 * H:2 * H + HH]], axis=1)
    xrmh_ref[...] = jnp.concatenate([x[:, H + HH:2 * H], x[:, 2 * H + HH:3 * H]], axis=1)


def _node_mlp(q, W1, b1, W2, b2, bn):
    n = q.shape[0]
    full = lambda a: pl.BlockSpec(a.shape, lambda i: (0,) * a.ndim)
    pair = jax.ShapeDtypeStruct((n, H), jnp.float32)
    return pl.pallas_call(
        _node_body,
        grid=(n // bn,),
        in_specs=[
            pl.BlockSpec((bn, H), lambda i: (i, 0)),
            full(W1), full(b1), full(W2), full(b2),
        ],
        out_specs=[pl.BlockSpec((bn, H), lambda i: (i, 0))] * 3,
        out_shape=[pair, pair, pair],
    )(q, W1, b1, W2, b2)


def _filter_body(rbf_ref, cut_ref, uv_ref, Wf1_ref, bf1_ref, Wf2_ref, bf2_ref,
                 fq_ref, g0l_ref, g0h_ref, g1l_ref, g1h_ref, g2l_ref, g2h_ref):
    h = _silu(jnp.dot(rbf_ref[...], Wf1_ref[...], preferred_element_type=jnp.float32)
              + bf1_ref[...])
    f = (jnp.dot(h, Wf2_ref[...], preferred_element_type=jnp.float32)
         + bf2_ref[...]) * cut_ref[...]
    fq_ref[...] = f[:, 0:H]
    frl = f[:, H:H + HH]
    frh = f[:, H + HH:2 * H]
    fml = f[:, 2 * H:2 * H + HH]
    fmh = f[:, 2 * H + HH:3 * H]
    for c, (gl_ref, gh_ref) in enumerate(((g0l_ref, g0h_ref),
                                          (g1l_ref, g1h_ref),
                                          (g2l_ref, g2h_ref))):
        uvc = uv_ref[:, c:c + 1]
        gl_ref[...] = jnp.concatenate([frl * uvc, fml], axis=1)
        gh_ref[...] = jnp.concatenate([frh * uvc, fmh], axis=1)


def _filter_mlp(rbf_p, cut_p, uv_p, Wf1, bf1, Wf2, bf2, be):
    ep, nrbf = rbf_p.shape
    full = lambda a: pl.BlockSpec(a.shape, lambda i: (0,) * a.ndim)
    pair = jax.ShapeDtypeStruct((ep, H), jnp.float32)
    return pl.pallas_call(
        _filter_body,
        grid=(ep // be,),
        in_specs=[
            pl.BlockSpec((be, nrbf), lambda i: (i, 0)),
            pl.BlockSpec((be, 1), lambda i: (i, 0)),
            pl.BlockSpec((be, 3), lambda i: (i, 0)),
            full(Wf1), full(bf1), full(Wf2), full(bf2),
        ],
        out_specs=[pl.BlockSpec((be, H), lambda i: (i, 0))] * 7,
        out_shape=[pair] * 7,
    )(rbf_p, cut_p, uv_p, Wf1, bf1, Wf2, bf2)


def _sc_messages(src2, tgt2, xq, xrm_l, xrm_h, mus_l, mus_h,
                 fq, gfs_l, gfs_h, zacc, zdeg, ones_hbm, np_rows):
    nblk = src2.shape[0]
    ep = nblk * B
    e_per_tile = ep // NTILES
    chunks = e_per_tile // B
    outers2 = chunks // (2 * K)
    blk_per_tile = chunks
    rpt = np_rows // NTILES

    mesh = plsc.VectorSubcoreMesh(core_axis_name="c", subcore_axis_name="s")
    acc_ty = jax.ShapeDtypeStruct((np_rows, HH), jnp.float32)
    deg_ty = jax.ShapeDtypeStruct((np_rows, 8), jnp.float32)

    @functools.partial(
        pl.kernel,
        out_type=[acc_ty, acc_ty, deg_ty,
                  acc_ty, acc_ty, acc_ty, acc_ty, acc_ty, acc_ty],
        mesh=mesh,
        scratch_types=[
            pltpu.VMEM_SHARED((np_rows, HH), jnp.float32),
            pltpu.VMEM_SHARED((np_rows, 8), jnp.float32),
            pltpu.VMEM((K, B), jnp.int32),
            pltpu.VMEM((K, B), jnp.int32),
            pltpu.VMEM((K, B), jnp.int32),
            pltpu.VMEM((K, B), jnp.int32),
            pltpu.VMEM((B, H), jnp.float32),
            pltpu.VMEM((B, H), jnp.float32),
            pltpu.VMEM((B, H), jnp.float32),
            pltpu.VMEM((B, H), jnp.float32),
            pltpu.VMEM((B, HH), jnp.float32),
            pltpu.VMEM((B, HH), jnp.float32),
            pltpu.VMEM((B, 8), jnp.float32),
            pltpu.SemaphoreType.DMA,
            pltpu.SemaphoreType.DMA,
            pltpu.SemaphoreType.DMA,
        ],
        compiler_params=pltpu.CompilerParams(use_tc_tiling_on_sc=False),
    )
    def sc_kernel(src2_h, tgt2_h, xq_h, xrml_h, xrmh_h,
                  mu0l_h, mu1l_h, mu2l_h, mu0h_h, mu1h_h, mu2h_h,
                  fq_h, gf0l_h, gf1l_h, gf2l_h, gf0h_h, gf1h_h, gf2h_h,
                  zacc_h, zdeg_h, ones_h,
                  oql_h, oqh_h, odeg_h, o0l_h, o0h_h, o1l_h, o1h_h, o2l_h, o2h_h,
                  acc_sh, deg_sh, idx_sA, idx_tA, idx_sB, idx_tB,
                  big_a0, big_a1, big_b0, big_b1, t64_0, t64_1,
                  ones, sem_a, sem_b, sem_c):
        cid = lax.axis_index("c")
        sid = lax.axis_index("s")
        row0 = sid * rpt
        big_a = (big_a0, big_a1)
        big_b = (big_b0, big_b1)
        t64 = (t64_0, t64_1)
        idx_s = (idx_sA, idx_sB)
        idx_t = (idx_tA, idx_tB)

        pltpu.sync_copy(ones_h, ones)

        def load_idx(which, blk):
            r0 = (sid * e_per_tile) // B + blk * K
            pltpu.sync_copy(src2_h.at[pl.ds(r0, K)], idx_s[which])
            pltpu.sync_copy(tgt2_h.at[pl.ds(r0, K)], idx_t[which])

        def zero_acc():
            pltpu.sync_copy(zacc_h.at[pl.ds(row0, rpt)],
                            acc_sh.at[pl.ds(row0, rpt)])

        def drain_acc(out_h):
            pltpu.sync_copy(acc_sh.at[pl.ds(row0, rpt)],
                            out_h.at[pl.ds(row0, rpt)])

        def make_sweep(fire, compute, do_deg):
            def fire_chunk(s, which_idx, k, blk):
                e0 = sid * e_per_tile + blk * (K * B) + k * B
                fire(s, idx_s[which_idx].at[k], e0)

            def process(s, which_idx, k):
                compute(s)
                pltpu.sync_copy(t64[s], acc_sh.at[idx_t[which_idx].at[k]],
                                add=True)
                if do_deg:
                    pltpu.sync_copy(ones, deg_sh.at[idx_t[which_idx].at[k]],
                                    add=True)

            def body(o2, carry):
                a = 2 * o2
                load_idx(1, a + 1)
                for k in range(K):
                    p = k % 2
                    if k < K - 1:
                        fire_chunk(1 - p, 0, k + 1, a)
                    else:
                        fire_chunk(1 - p, 1, 0, a + 1)
                    process(p, 0, k)

                @pl.when(o2 < outers2 - 1)
                def _():
                    load_idx(0, a + 2)
                for k in range(K):
                    p = k % 2
                    if k < K - 1:
                        fire_chunk(1 - p, 1, k + 1, a + 1)
                    else:
                        @pl.when(o2 < outers2 - 1)
                        def _():
                            fire_chunk(1 - p, 0, 0, a + 2)
                    process(p, 1, k)
                return carry

            load_idx(0, 0)
            fire_chunk(0, 0, 0, 0)
            lax.fori_loop(0, outers2, body, 0)

        def sweep_vec(xrm_h, mu_h, gf_h):
            def fire(s, idx_row, e0):
                pltpu.async_copy(xrm_h.at[idx_row], big_a[s], sem_a)
                pltpu.async_copy(mu_h.at[idx_row], t64[s], sem_b)
                pltpu.async_copy(gf_h.at[pl.ds(e0, B)], big_b[s], sem_c)

            def compute(s):
                pltpu.make_async_copy(xrm_h, big_a[s], sem_a).wait()
                pltpu.make_async_copy(mu_h, t64[s], sem_b).wait()
                pltpu.make_async_copy(gf_h, big_b[s], sem_c).wait()
                ba, bb, t = big_a[s], big_b[s], t64[s]

                def row(r, c):
                    for gi in range(HH // 16):
                        lo = pl.ds(gi * 16, 16)
                        hi = pl.ds(HH + gi * 16, 16)
                        t[r, lo] = (ba[r, lo] * bb[r, lo]
                                    + t[r, lo] * (ba[r, hi] * bb[r, hi]))
                    return c
                lax.fori_loop(0, B, row, 0)

            make_sweep(fire, compute, do_deg=False)

        def sweep_q(half, do_deg):
            def fire(s, idx_row, e0):
                pltpu.async_copy(xq_h.at[idx_row], big_a[s], sem_a)
                pltpu.async_copy(fq_h.at[pl.ds(e0, B)], big_b[s], sem_b)

            def compute(s):
                pltpu.make_async_copy(xq_h, big_a[s], sem_a).wait()
                pltpu.make_async_copy(fq_h, big_b[s], sem_b).wait()
                ba, bb, t = big_a[s], big_b[s], t64[s]

                def row(r, c):
                    for gi in range(HH // 16):
                        lo = pl.ds(gi * 16, 16)
                        sh = pl.ds(half + gi * 16, 16)
                        t[r, lo] = ba[r, sh] * bb[r, sh]
                    return c
                lax.fori_loop(0, B, row, 0)

            make_sweep(fire, compute, do_deg)

        def program(half, xrm_h, mu_hs, gf_hs, oq_h, ov_hs, do_deg):
            zero_acc()
            if do_deg:
                pltpu.sync_copy(zdeg_h.at[pl.ds(row0, rpt)],
                                deg_sh.at[pl.ds(row0, rpt)])
            plsc.subcore_barrier()
            sweep_q(half, do_deg)
            plsc.subcore_barrier()
            drain_acc(oq_h)
            if do_deg:
                pltpu.sync_copy(deg_sh.at[pl.ds(row0, rpt)],
                                odeg_h.at[pl.ds(row0, rpt)])
            for c in range(3):
                zero_acc()
                plsc.subcore_barrier()
                sweep_vec(xrm_h, mu_hs[c], gf_hs[c])
                plsc.subcore_barrier()
                drain_acc(ov_hs[c])

        @pl.when(cid == 0)
        def _():
            program(0, xrml_h, (mu0l_h, mu1l_h, mu2l_h),
                    (gf0l_h, gf1l_h, gf2l_h), oql_h, (o0l_h, o1l_h, o2l_h),
                    do_deg=True)

        @pl.when(cid == 1)
        def _():
            program(HH, xrmh_h, (mu0h_h, mu1h_h, mu2h_h),
                    (gf0h_h, gf1h_h, gf2h_h), oqh_h, (o0h_h, o1h_h, o2h_h),
                    do_deg=False)

    return sc_kernel(src2, tgt2, xq, xrm_l, xrm_h,
                     mus_l[0], mus_l[1], mus_l[2], mus_h[0], mus_h[1], mus_h[2],
                     fq, gfs_l[0], gfs_l[1], gfs_l[2],
                     gfs_h[0], gfs_h[1], gfs_h[2], zacc, zdeg, ones_hbm)


def _final_body(q_ref, mu_ref, aql_ref, aqh_ref, adeg_ref,
                a0l_ref, a0h_ref, a1l_ref, a1h_ref, a2l_ref, a2h_ref,
                Wv_ref, Wm1_ref, bm1_ref, Wm2_ref, bm2_ref,
                qo_ref, muo_ref):
    deg = jnp.clip(adeg_ref[:, 0:1], 1.0, None)
    inv = 1.0 / deg
    qn = q_ref[...] + jnp.concatenate([aql_ref[...], aqh_ref[...]], axis=1) * inv
    avs = ((a0l_ref[...], a0h_ref[...]), (a1l_ref[...], a1h_ref[...]),
           (a2l_ref[...], a2h_ref[...]))
    mun = [mu_ref[:, c, :] + jnp.concatenate(avs[c], axis=1) * inv
           for c in range(3)]
    mc = [jnp.dot(m, Wv_ref[...], preferred_element_type=jnp.float32) for m in mun]
    mu_v = [m[:, :H] for m in mc]
    mu_w = [m[:, H:] for m in mc]
    nrm = jnp.sqrt(mu_v[0] ** 2 + mu_v[1] ** 2 + mu_v[2] ** 2 + 1e-08)
    si = jnp.concatenate([qn, nrm], axis=-1)
    delta = (jnp.dot(_silu(jnp.dot(si, Wm1_ref[...], preferred_element_type=jnp.float32)
                           + bm1_ref[...]),
                     Wm2_ref[...], preferred_element_type=jnp.float32) + bm2_ref[...])
    dq = delta[:, :H]
    dmu = delta[:, H:2 * H]
    dqmu = delta[:, 2 * H:]
    inner = mu_v[0] * mu_w[0] + mu_v[1] * mu_w[1] + mu_v[2] * mu_w[2]
    qo_ref[...] = qn + dq + dqmu * inner
    muo_ref[...] = jnp.stack([mun[c] + mu_w[c] * dmu for c in range(3)], axis=1)


def _final_mix(q, mu, aq_l, aq_h, adeg, avs_l, avs_h, Wv, Wm1, bm1, Wm2, bm2, bn):
    n = q.shape[0]
    full = lambda a: pl.BlockSpec(a.shape, lambda i: (0,) * a.ndim)
    row = lambda w: pl.BlockSpec((bn, w), lambda i: (i, 0))
    return pl.pallas_call(
        _final_body,
        grid=(n // bn,),
        in_specs=[
            row(H),
            pl.BlockSpec((bn, 3, H), lambda i: (i, 0, 0)),
            row(HH), row(HH), row(8),
            row(HH), row(HH), row(HH), row(HH), row(HH), row(HH),
            full(Wv), full(Wm1), full(bm1), full(Wm2), full(bm2),
        ],
        out_specs=[
            row(H),
            pl.BlockSpec((bn, 3, H), lambda i: (i, 0, 0)),
        ],
        out_shape=[
            jax.ShapeDtypeStruct((n, H), jnp.float32),
            jax.ShapeDtypeStruct((n, 3, H), jnp.float32),
        ],
    )(q, mu, aq_l, aq_h, adeg, avs_l[0], avs_h[0], avs_l[1], avs_h[1],
      avs_l[2], avs_h[2], Wv, Wm1, bm1, Wm2, bm2)


def kernel(q, mu, edge_index, rbf, unit_vectors, cutoff_values,
           W1, b1, W2, b2, Wf1, bf1, Wf2, bf2, Wv, Wm1, bm1, Wm2, bm2):
    n = q.shape[0]
    e = edge_index.shape[1]

    ew = NTILES * B * K * 2
    ep = ((e + ew - 1) // ew) * ew
    np_rows = ((n + 8) + NTILES * 8 - 1) // (NTILES * 8) * (NTILES * 8)

    source = edge_index[1]
    target = edge_index[0]
    pad = ep - e
    src2 = jnp.concatenate([source, jnp.zeros((pad,), jnp.int32)]).reshape(ep // B, B)
    tgt2 = jnp.concatenate([target, jnp.full((pad,), n, jnp.int32)]).reshape(ep // B, B)
    rbf_p = jnp.pad(rbf, ((0, pad), (0, 0)))
    cut_p = jnp.pad(cutoff_values, (0, pad))[:, None]
    uv_p = jnp.pad(unit_vectors, ((0, pad), (0, 0)))
    mu_t = jnp.transpose(mu, (1, 0, 2))
    mus_l = tuple(mu_t[c, :, :HH] for c in range(3))
    mus_h = tuple(mu_t[c, :, HH:] for c in range(3))
    zacc = jnp.zeros((np_rows, HH), jnp.float32)
    zdeg = jnp.zeros((np_rows, 8), jnp.float32)
    ones_hbm = jnp.ones((B, 8), jnp.float32)

    b1r, b2r = b1[None, :], b2[None, :]
    bf1r, bf2r = bf1[None, :], bf2[None, :]
    bm1r, bm2r = bm1[None, :], bm2[None, :]

    xq, xrm_l, xrm_h = _node_mlp(q, W1, b1r, W2, b2r, bn=1000)
    fq, gf0_l, gf0_h, gf1_l, gf1_h, gf2_l, gf2_h = _filter_mlp(
        rbf_p, cut_p, uv_p, Wf1, bf1r, Wf2, bf2r, be=1024)
    aq_l, aq_h, adeg, a0_l, a0_h, a1_l, a1_h, a2_l, a2_h = _sc_messages(
        src2, tgt2, xq, xrm_l, xrm_h, mus_l, mus_h,
        fq, (gf0_l, gf1_l, gf2_l), (gf0_h, gf1_h, gf2_h),
        zacc, zdeg, ones_hbm, np_rows)
    return _final_mix(q, mu, aq_l, aq_h, adeg, (a0_l, a1_l, a2_l),
                      (a0_h, a1_h, a2_h), Wv, Wm1, bm1r, Wm2, bm2r, bn=1000)
